# baseline (speedup 1.0000x reference)
import functools

import jax
import jax.numpy as jnp
from jax import lax
from jax.experimental import pallas as pl
from jax.experimental.pallas import tpu as pltpu
from jax.experimental.pallas import tpu_sc as plsc

N = 10000
E = 320000
SVG = 128
PCAP = 64
H = 128
NPROC = 128
NIPS = 1024
G = 64

NC = 2
NS = 16
NW = NC * NS

C = 128
EPW = E // NW
NCHF = EPW // C
CT = EPW - NCHF * C

NP = 10240
RPT = NP // NS


def _mesh():
  return plsc.VectorSubcoreMesh(
      core_axis_name="c", subcore_axis_name="s", num_cores=NC, num_subcores=NS
  )


def _deg_body(dst_hbm, out_hbm, didx0, didx1, didxt, ones_v, zeros_v, acc,
              semi0, semi1):
  cid = lax.axis_index("c")
  sid = lax.axis_index("s")
  wid = cid * NS + sid
  ebase = wid * EPW

  def load_idx(j, didx, semi):
    pltpu.async_copy(dst_hbm.at[pl.ds(ebase + j * C, C)], didx, semi)

  def wait_idx(didx, semi):
    pltpu.make_async_copy(dst_hbm.at[pl.ds(ebase, C)], didx, semi).wait()

  load_idx(0, didx0, semi0)
  load_idx(1, didx1, semi1)

  def fill(i, carry):
    zeros_v[pl.ds(i * 16, 16)] = jnp.zeros((16,), jnp.float32)
    return carry

  lax.fori_loop(0, RPT // 16, fill, 0)

  def fill1(i, carry):
    ones_v[pl.ds(i * 16, 16)] = jnp.ones((16,), jnp.float32)
    return carry

  lax.fori_loop(0, C // 16, fill1, 0)

  pltpu.sync_copy(zeros_v, acc.at[pl.ds(sid * RPT, RPT)])
  plsc.subcore_barrier()

  def body(i, carry):
    j = 2 * i
    wait_idx(didx0, semi0)
    pltpu.sync_copy(ones_v, acc.at[didx0], add=True)
    load_idx(j + 2, didx0, semi0)
    wait_idx(didx1, semi1)
    pltpu.sync_copy(ones_v, acc.at[didx1], add=True)
    load_idx(j + 3, didx1, semi1)
    return carry

  lax.fori_loop(0, NCHF // 2 - 1, body, 0)
  pltpu.async_copy(dst_hbm.at[pl.ds(ebase + NCHF * C, CT)], didxt, semi0)
  wait_idx(didx0, semi0)
  pltpu.sync_copy(ones_v, acc.at[didx0], add=True)
  wait_idx(didx1, semi1)
  pltpu.sync_copy(ones_v, acc.at[didx1], add=True)
  pltpu.make_async_copy(
      dst_hbm.at[pl.ds(ebase, CT)], didxt, semi0).wait()
  pltpu.sync_copy(ones_v.at[pl.ds(0, CT)], acc.at[didxt], add=True)

  plsc.subcore_barrier()
  pltpu.sync_copy(acc.at[pl.ds(sid * RPT, RPT)],
                  out_hbm.at[pl.ds(cid * NP + sid * RPT, RPT)])


@functools.cache
def _deg_call():
  return pl.kernel(
      _deg_body,
      out_type=jax.ShapeDtypeStruct((NC * NP,), jnp.float32),
      mesh=_mesh(),
      scratch_types=[
          pltpu.VMEM((C,), jnp.int32),
          pltpu.VMEM((C,), jnp.int32),
          pltpu.VMEM((CT,), jnp.int32),
          pltpu.VMEM((C,), jnp.float32),
          pltpu.VMEM((RPT,), jnp.float32),
          pltpu.VMEM_SHARED((NP,), jnp.float32),
          pltpu.SemaphoreType.DMA,
          pltpu.SemaphoreType.DMA,
      ],
  )


def _scat_body(src_hbm, dst_hbm, table_hbm, zeros_hbm, out_hbm,
               sidx0, didx0, sidx1, didx1, sidxt, didxt, rows0, rows1, rowst,
               acc, semi0, semi1, sem0, sem1):
  cid = lax.axis_index("c")
  sid = lax.axis_index("s")
  wid = cid * NS + sid
  ebase = wid * EPW

  def load_idx(j, sidx, didx, semi):
    pltpu.async_copy(src_hbm.at[pl.ds(ebase + j * C, C)], sidx, semi)
    pltpu.async_copy(dst_hbm.at[pl.ds(ebase + j * C, C)], didx, semi)

  def wait_idx(sidx, didx, semi):
    pltpu.make_async_copy(src_hbm.at[pl.ds(ebase, C)], sidx, semi).wait()
    pltpu.make_async_copy(dst_hbm.at[pl.ds(ebase, C)], didx, semi).wait()

  load_idx(0, sidx0, didx0, semi0)
  load_idx(1, sidx1, didx1, semi1)
  pltpu.sync_copy(zeros_hbm, acc.at[pl.ds(sid * RPT, RPT)])
  wait_idx(sidx0, didx0, semi0)
  plsc.subcore_barrier()
  pltpu.async_copy(table_hbm.at[sidx0], rows0, sem0)

  def half(j, sidx_a, didx_a, semi_a, rows_a, sem_a,
           sidx_b, didx_b, semi_b, rows_b, sem_b):
    wait_idx(sidx_b, didx_b, semi_b)
    pltpu.async_copy(table_hbm.at[sidx_b], rows_b, sem_b)
    pltpu.make_async_copy(table_hbm.at[sidx_a], rows_a, sem_a).wait()
    pltpu.sync_copy(rows_a, acc.at[didx_a], add=True)
    load_idx(j + 2, sidx_a, didx_a, semi_a)

  def body(i, carry):
    j = 2 * i
    half(j, sidx0, didx0, semi0, rows0, sem0,
         sidx1, didx1, semi1, rows1, sem1)
    half(j + 1, sidx1, didx1, semi1, rows1, sem1,
         sidx0, didx0, semi0, rows0, sem0)
    return carry

  lax.fori_loop(0, NCHF // 2 - 1, body, 0)
  wait_idx(sidx1, didx1, semi1)
  pltpu.async_copy(table_hbm.at[sidx1], rows1, sem1)
  pltpu.async_copy(src_hbm.at[pl.ds(ebase + NCHF * C, CT)], sidxt, semi0)
  pltpu.async_copy(dst_hbm.at[pl.ds(ebase + NCHF * C, CT)], didxt, semi0)
  pltpu.make_async_copy(table_hbm.at[sidx0], rows0, sem0).wait()
  pltpu.sync_copy(rows0, acc.at[didx0], add=True)
  pltpu.make_async_copy(src_hbm.at[pl.ds(ebase, CT)], sidxt, semi0).wait()
  pltpu.make_async_copy(dst_hbm.at[pl.ds(ebase, CT)], didxt, semi0).wait()
  pltpu.async_copy(table_hbm.at[sidxt], rowst, sem0)
  pltpu.make_async_copy(table_hbm.at[sidx1], rows1, sem1).wait()
  pltpu.sync_copy(rows1, acc.at[didx1], add=True)
  pltpu.make_async_copy(table_hbm.at[sidxt], rowst, sem0).wait()
  pltpu.sync_copy(rowst, acc.at[didxt], add=True)

  plsc.subcore_barrier()
  pltpu.sync_copy(acc.at[pl.ds(sid * RPT, RPT)],
                  out_hbm.at[cid, pl.ds(sid * RPT, RPT)])


@functools.cache
def _scat_call():
  return pl.kernel(
      _scat_body,
      out_type=jax.ShapeDtypeStruct((NC, NP, H), jnp.float32),
      mesh=_mesh(),
      scratch_types=[
          pltpu.VMEM((C,), jnp.int32),
          pltpu.VMEM((C,), jnp.int32),
          pltpu.VMEM((C,), jnp.int32),
          pltpu.VMEM((C,), jnp.int32),
          pltpu.VMEM((CT,), jnp.int32),
          pltpu.VMEM((CT,), jnp.int32),
          pltpu.VMEM((C, H), jnp.float32),
          pltpu.VMEM((C, H), jnp.float32),
          pltpu.VMEM((CT, H), jnp.float32),
          pltpu.VMEM_SHARED((NP, H), jnp.float32),
          pltpu.SemaphoreType.DMA,
          pltpu.SemaphoreType.DMA,
          pltpu.SemaphoreType.DMA,
          pltpu.SemaphoreType.DMA,
      ],
  )


def _dinv(d0_ref, d1_ref):
  return lax.rsqrt(d0_ref[...] + d1_ref[...] + 1.0)


def _tc1_body(x_ref, w1_ref, d0_ref, d1_ref, hs1_ref):
  h = jnp.dot(x_ref[...], w1_ref[...], preferred_element_type=jnp.float32)
  hs1_ref[...] = h * _dinv(d0_ref, d1_ref)


def _tc1_call(x, w1, d0, d1):
  return pl.pallas_call(
      _tc1_body,
      out_shape=jax.ShapeDtypeStruct((N, H), jnp.float32),
  )(x, w1, d0, d1)


def _tc2_body(agg_ref, hs1_ref, d0_ref, d1_ref, w2_ref, b1_ref, hs2_ref):
  dinv = _dinv(d0_ref, d1_ref)
  agg = agg_ref[0, :N, :] + agg_ref[1, :N, :]
  out1 = (agg + hs1_ref[...]) * dinv + b1_ref[...]
  h1 = jnp.maximum(out1, 0.0)
  hs2_ref[...] = jnp.dot(h1, w2_ref[...],
                         preferred_element_type=jnp.float32) * dinv


def _tc2_call(agg1, hs1, d0, d1, w2, b1r):
  return pl.pallas_call(
      _tc2_body,
      out_shape=jax.ShapeDtypeStruct((N, H), jnp.float32),
  )(agg1, hs1, d0, d1, w2, b1r)


def _tc3_body(agg_ref, hs2_ref, d0_ref, d1_ref, b2_ref, batch_ref, pcap_ref,
              wc_ref, bc_ref, wot_ref, bo_ref, wpt_ref, bp_ref,
              orig_ref, proc_ref):
  dinv = _dinv(d0_ref, d1_ref)
  agg = agg_ref[0, :N, :] + agg_ref[1, :N, :]
  h2 = (agg + hs2_ref[...]) * dinv + b2_ref[...]
  ids = lax.broadcasted_iota(jnp.int32, (G, N), 0)
  oh = jnp.where(batch_ref[...] == ids, 1.0, 0.0)
  sums = jnp.dot(oh, h2, preferred_element_type=jnp.float32)
  counts = jnp.sum(oh, axis=1, keepdims=True)
  ge = sums / jnp.maximum(counts, 1.0)
  pe = jnp.dot(pcap_ref[...], wc_ref[...],
               preferred_element_type=jnp.float32) + bc_ref[...]
  comb = jnp.concatenate([ge, pe], axis=1)
  orig_ref[...] = jnp.dot(comb, wot_ref[...],
                          preferred_element_type=jnp.float32) + bo_ref[...]
  proc_ref[...] = jnp.dot(comb, wpt_ref[...],
                          preferred_element_type=jnp.float32) + bp_ref[...]


def _tc3_call(agg2, hs2, d0, d1, b2r, batch_r, pcap, wc, bcr, wot, bor, wpt,
              bpr):
  return pl.pallas_call(
      _tc3_body,
      out_shape=[
          jax.ShapeDtypeStruct((G, NIPS), jnp.float32),
          jax.ShapeDtypeStruct((G, NPROC), jnp.float32),
      ],
  )(agg2, hs2, d0, d1, b2r, batch_r, pcap, wc, bcr, wot, bor, wpt, bpr)


@jax.jit
def kernel(x, edge_index, batch, pcap_features, W1, b1, W2, b2, Wc, bc,
           Wo, bo, Wp, bp):
  src = edge_index[0]
  dst = edge_index[1]
  zeros_h = jnp.zeros((RPT, H), jnp.float32)

  degf = _deg_call()(dst)
  d0 = degf[:N].reshape(N, 1)
  d1 = degf[NP:NP + N].reshape(N, 1)
  hs1 = _tc1_call(x, W1, d0, d1)
  agg1 = _scat_call()(src, dst, hs1, zeros_h)
  hs2 = _tc2_call(agg1, hs1, d0, d1, W2, b1[None, :])
  agg2 = _scat_call()(src, dst, hs2, zeros_h)
  origin, process = _tc3_call(
      agg2, hs2, d0, d1, b2[None, :], batch.reshape(1, N), pcap_features,
      Wc[:, :, 1].T, bc[None, :], Wo.T, bo[None, :], Wp.T, bp[None, :])
  return (origin, process)


# speedup vs baseline: 1.0267x; 1.0267x over previous
import functools

import jax
import jax.numpy as jnp
from jax import lax
from jax.experimental import pallas as pl
from jax.experimental.pallas import tpu as pltpu
from jax.experimental.pallas import tpu_sc as plsc

N = 10000
E = 320000
SVG = 128
PCAP = 64
H = 128
NPROC = 128
NIPS = 1024
G = 64

NC = 2
NS = 16
NW = NC * NS

C = 128
C2 = 64
EPW = E // NW
NCHF = EPW // C
NCHF2 = EPW // C2
CT = EPW - NCHF * C

NP = 10240
RPT = NP // NS


def _mesh():
  return plsc.VectorSubcoreMesh(
      core_axis_name="c", subcore_axis_name="s", num_cores=NC, num_subcores=NS
  )


def _deg_body(dst_hbm, out_hbm, didx_all, ones_v, zeros_v, acc, sem):
  cid = lax.axis_index("c")
  sid = lax.axis_index("s")
  wid = cid * NS + sid
  ebase = wid * EPW
  pltpu.async_copy(dst_hbm.at[pl.ds(ebase, EPW)], didx_all, sem)

  def fill(i, carry):
    zeros_v[pl.ds(i * 16, 16)] = jnp.zeros((16,), jnp.float32)
    return carry

  lax.fori_loop(0, RPT // 16, fill, 0)

  def fill1(i, carry):
    ones_v[pl.ds(i * 16, 16)] = jnp.ones((16,), jnp.float32)
    return carry

  lax.fori_loop(0, C // 16, fill1, 0)

  pltpu.sync_copy(zeros_v, acc.at[pl.ds(sid * RPT, RPT)])
  pltpu.make_async_copy(dst_hbm.at[pl.ds(0, EPW)], didx_all, sem).wait()
  plsc.subcore_barrier()

  def body(i, carry):
    pltpu.sync_copy(ones_v, acc.at[didx_all.at[pl.ds(i * C, C)]], add=True)
    return carry

  lax.fori_loop(0, NCHF, body, 0)
  pltpu.sync_copy(ones_v.at[pl.ds(0, CT)],
                  acc.at[didx_all.at[pl.ds(NCHF * C, CT)]], add=True)

  plsc.subcore_barrier()
  pltpu.sync_copy(acc.at[pl.ds(sid * RPT, RPT)],
                  out_hbm.at[pl.ds(cid * NP + sid * RPT, RPT)])


@functools.cache
def _deg_call():
  return pl.kernel(
      _deg_body,
      out_type=jax.ShapeDtypeStruct((NC * NP,), jnp.float32),
      mesh=_mesh(),
      scratch_types=[
          pltpu.VMEM((EPW,), jnp.int32),
          pltpu.VMEM((C,), jnp.float32),
          pltpu.VMEM((RPT,), jnp.float32),
          pltpu.VMEM_SHARED((NP,), jnp.float32),
          pltpu.SemaphoreType.DMA,
      ],
  )


def _scat_body(src_hbm, dst_hbm, table_hbm, zeros_hbm, out_hbm,
               sidx_all, didx_all, rows0, rows1, rowst, acc,
               semi, sem0, sem1):
  cid = lax.axis_index("c")
  sid = lax.axis_index("s")
  wid = cid * NS + sid
  ebase = wid * EPW

  pltpu.async_copy(src_hbm.at[pl.ds(ebase, EPW)], sidx_all, semi)
  pltpu.async_copy(dst_hbm.at[pl.ds(ebase, EPW)], didx_all, semi)
  pltpu.sync_copy(zeros_hbm, acc.at[pl.ds(sid * RPT, RPT)])
  pltpu.make_async_copy(src_hbm.at[pl.ds(0, EPW)], sidx_all, semi).wait()
  pltpu.make_async_copy(dst_hbm.at[pl.ds(0, EPW)], didx_all, semi).wait()
  plsc.subcore_barrier()

  def sidx(j):
    return sidx_all.at[pl.ds(j * C2, C2)]

  def didx(j):
    return didx_all.at[pl.ds(j * C2, C2)]

  pltpu.async_copy(table_hbm.at[sidx(0)], rows0, sem0)

  def half(j, rows_a, sem_a, rows_b, sem_b):
    pltpu.async_copy(table_hbm.at[sidx(j + 1)], rows_b, sem_b)
    pltpu.make_async_copy(table_hbm.at[sidx(j)], rows_a, sem_a).wait()
    pltpu.sync_copy(rows_a, acc.at[didx(j)], add=True)

  def body(i, carry):
    j = 2 * i
    half(j, rows0, sem0, rows1, sem1)
    half(j + 1, rows1, sem1, rows0, sem0)
    return carry

  lax.fori_loop(0, NCHF2 // 2 - 1, body, 0)
  j0 = NCHF2 - 2
  pltpu.async_copy(table_hbm.at[sidx(j0 + 1)], rows1, sem1)
  pltpu.make_async_copy(table_hbm.at[sidx(j0)], rows0, sem0).wait()
  pltpu.sync_copy(rows0, acc.at[didx(j0)], add=True)
  pltpu.async_copy(
      table_hbm.at[sidx_all.at[pl.ds(NCHF2 * C2, CT)]], rowst, sem0)
  pltpu.make_async_copy(table_hbm.at[sidx(j0 + 1)], rows1, sem1).wait()
  pltpu.sync_copy(rows1, acc.at[didx(j0 + 1)], add=True)
  pltpu.make_async_copy(
      table_hbm.at[sidx_all.at[pl.ds(NCHF2 * C2, CT)]], rowst, sem0).wait()
  pltpu.sync_copy(rowst, acc.at[didx_all.at[pl.ds(NCHF2 * C2, CT)]], add=True)

  plsc.subcore_barrier()
  pltpu.sync_copy(acc.at[pl.ds(sid * RPT, RPT)],
                  out_hbm.at[cid, pl.ds(sid * RPT, RPT)])


@functools.cache
def _scat_call():
  return pl.kernel(
      _scat_body,
      out_type=jax.ShapeDtypeStruct((NC, NP, H), jnp.float32),
      mesh=_mesh(),
      scratch_types=[
          pltpu.VMEM((EPW,), jnp.int32),
          pltpu.VMEM((EPW,), jnp.int32),
          pltpu.VMEM((C2, H), jnp.float32),
          pltpu.VMEM((C2, H), jnp.float32),
          pltpu.VMEM((CT, H), jnp.float32),
          pltpu.VMEM_SHARED((NP, H), jnp.float32),
          pltpu.SemaphoreType.DMA,
          pltpu.SemaphoreType.DMA,
          pltpu.SemaphoreType.DMA,
      ],
  )


def _dinv(d_ref):
  return lax.rsqrt(d_ref[...] + 1.0)


def _tc1_body(x_ref, w1_ref, d_ref, hs1_ref):
  h = jnp.dot(x_ref[...], w1_ref[...], preferred_element_type=jnp.float32)
  hs1_ref[...] = h * _dinv(d_ref)


def _tc1_call(x, w1, d):
  return pl.pallas_call(
      _tc1_body,
      out_shape=jax.ShapeDtypeStruct((N, H), jnp.float32),
  )(x, w1, d)


def _tc2_body(agg_ref, hs1_ref, d_ref, w2_ref, b1_ref, hs2_ref):
  dinv = _dinv(d_ref)
  agg = agg_ref[0, :N, :] + agg_ref[1, :N, :]
  out1 = (agg + hs1_ref[...]) * dinv + b1_ref[...]
  h1 = jnp.maximum(out1, 0.0)
  hs2_ref[...] = jnp.dot(h1, w2_ref[...],
                         preferred_element_type=jnp.float32) * dinv


def _tc2_call(agg1, hs1, d, w2, b1r):
  return pl.pallas_call(
      _tc2_body,
      out_shape=jax.ShapeDtypeStruct((N, H), jnp.float32),
  )(agg1, hs1, d, w2, b1r)


def _tc3_body(agg_ref, hs2_ref, d_ref, b2_ref, batch_ref, pcap_ref,
              wc_ref, bc_ref, wot_ref, bo_ref, wpt_ref, bp_ref,
              orig_ref, proc_ref):
  dinv = _dinv(d_ref)
  agg = agg_ref[0, :N, :] + agg_ref[1, :N, :]
  h2 = (agg + hs2_ref[...]) * dinv + b2_ref[...]
  ids = lax.broadcasted_iota(jnp.int32, (G, N), 0)
  oh = jnp.where(batch_ref[...] == ids, 1.0, 0.0)
  sums = jnp.dot(oh, h2, preferred_element_type=jnp.float32)
  counts = jnp.sum(oh, axis=1, keepdims=True)
  ge = sums / jnp.maximum(counts, 1.0)
  pe = jnp.dot(pcap_ref[...], wc_ref[...],
               preferred_element_type=jnp.float32) + bc_ref[...]
  comb = jnp.concatenate([ge, pe], axis=1)
  orig_ref[...] = jnp.dot(comb, wot_ref[...],
                          preferred_element_type=jnp.float32) + bo_ref[...]
  proc_ref[...] = jnp.dot(comb, wpt_ref[...],
                          preferred_element_type=jnp.float32) + bp_ref[...]


def _tc3_call(agg2, hs2, d, b2r, batch_r, pcap, wc, bcr, wot, bor, wpt,
              bpr):
  return pl.pallas_call(
      _tc3_body,
      out_shape=[
          jax.ShapeDtypeStruct((G, NIPS), jnp.float32),
          jax.ShapeDtypeStruct((G, NPROC), jnp.float32),
      ],
  )(agg2, hs2, d, b2r, batch_r, pcap, wc, bcr, wot, bor, wpt, bpr)


@jax.jit
def kernel(x, edge_index, batch, pcap_features, W1, b1, W2, b2, Wc, bc,
           Wo, bo, Wp, bp):
  src = edge_index[0]
  dst = edge_index[1]
  zeros_h = jnp.zeros((RPT, H), jnp.float32)

  degf = _deg_call()(dst)
  d = (degf[:N] + degf[NP:NP + N]).reshape(N, 1)
  hs1 = _tc1_call(x, W1, d)
  agg1 = _scat_call()(src, dst, hs1, zeros_h)
  hs2 = _tc2_call(agg1, hs1, d, W2, b1[None, :])
  agg2 = _scat_call()(src, dst, hs2, zeros_h)
  origin, process = _tc3_call(
      agg2, hs2, d, b2[None, :], batch.reshape(1, N), pcap_features,
      Wc[:, :, 1].T, bc[None, :], Wo.T, bo[None, :], Wp.T, bp[None, :])
  return (origin, process)


# speedup vs baseline: 1.0753x; 1.0474x over previous
import functools

import jax
import jax.numpy as jnp
from jax import lax
from jax.experimental import pallas as pl
from jax.experimental.pallas import tpu as pltpu
from jax.experimental.pallas import tpu_sc as plsc

N = 10000
E = 320000
SVG = 128
PCAP = 64
H = 128
NPROC = 128
NIPS = 1024
G = 64

NC = 2
NS = 16
NW = NC * NS

C = 128
EPW = E // NW
NCHF = EPW // C
CT = EPW - NCHF * C

NP = 10240
RPT = NP // NS


def _mesh():
  return plsc.VectorSubcoreMesh(
      core_axis_name="c", subcore_axis_name="s", num_cores=NC, num_subcores=NS
  )


def _deg_body(dst_hbm, out_hbm, didx_all, ones_v, zeros_v, acc, sem):
  cid = lax.axis_index("c")
  sid = lax.axis_index("s")
  wid = cid * NS + sid
  ebase = wid * EPW
  pltpu.async_copy(dst_hbm.at[pl.ds(ebase, EPW)], didx_all, sem)

  def fill(i, carry):
    zeros_v[pl.ds(i * 16, 16)] = jnp.zeros((16,), jnp.float32)
    return carry

  lax.fori_loop(0, RPT // 16, fill, 0)

  def fill1(i, carry):
    ones_v[pl.ds(i * 16, 16)] = jnp.ones((16,), jnp.float32)
    return carry

  lax.fori_loop(0, C // 16, fill1, 0)

  pltpu.sync_copy(zeros_v, acc.at[pl.ds(sid * RPT, RPT)])
  pltpu.make_async_copy(dst_hbm.at[pl.ds(0, EPW)], didx_all, sem).wait()
  plsc.subcore_barrier()

  def body(i, carry):
    pltpu.sync_copy(ones_v, acc.at[didx_all.at[pl.ds(i * C, C)]], add=True)
    return carry

  lax.fori_loop(0, NCHF, body, 0)
  pltpu.sync_copy(ones_v.at[pl.ds(0, CT)],
                  acc.at[didx_all.at[pl.ds(NCHF * C, CT)]], add=True)

  plsc.subcore_barrier()
  pltpu.sync_copy(acc.at[pl.ds(sid * RPT, RPT)],
                  out_hbm.at[pl.ds(cid * NP + sid * RPT, RPT)])


@functools.cache
def _deg_call():
  return pl.kernel(
      _deg_body,
      out_type=jax.ShapeDtypeStruct((NC * NP,), jnp.float32),
      mesh=_mesh(),
      scratch_types=[
          pltpu.VMEM((EPW,), jnp.int32),
          pltpu.VMEM((C,), jnp.float32),
          pltpu.VMEM((RPT,), jnp.float32),
          pltpu.VMEM_SHARED((NP,), jnp.float32),
          pltpu.SemaphoreType.DMA,
      ],
  )


def _scat_body(src_hbm, dst_hbm, table_hbm, zeros_hbm, out_hbm,
               sidx0, sidx1, sidxt, didx_all, rows0, rows1, rowst, acc,
               semi0, semi1, sem0, sem1):
  cid = lax.axis_index("c")
  sid = lax.axis_index("s")
  wid = cid * NS + sid
  ebase = wid * EPW

  def load_sidx(j, sidx, semi):
    pltpu.async_copy(src_hbm.at[pl.ds(ebase + j * C, C)], sidx, semi)

  def wait_sidx(sidx, semi):
    pltpu.make_async_copy(src_hbm.at[pl.ds(ebase, C)], sidx, semi).wait()

  def didx(j):
    return didx_all.at[pl.ds(j * C, C)]

  pltpu.async_copy(dst_hbm.at[pl.ds(ebase, EPW)], didx_all, semi0)
  load_sidx(0, sidx0, semi0)
  load_sidx(1, sidx1, semi1)
  pltpu.sync_copy(zeros_hbm, acc.at[pl.ds(sid * RPT, RPT)])
  pltpu.make_async_copy(dst_hbm.at[pl.ds(0, EPW)], didx_all, semi0).wait()
  wait_sidx(sidx0, semi0)
  plsc.subcore_barrier()
  pltpu.async_copy(table_hbm.at[sidx0], rows0, sem0)

  def half(j, sidx_a, semi_a, rows_a, sem_a, sidx_b, semi_b, rows_b, sem_b):
    wait_sidx(sidx_b, semi_b)
    pltpu.async_copy(table_hbm.at[sidx_b], rows_b, sem_b)
    pltpu.make_async_copy(table_hbm.at[sidx_a], rows_a, sem_a).wait()
    pltpu.sync_copy(rows_a, acc.at[didx(j)], add=True)
    load_sidx(j + 2, sidx_a, semi_a)

  def body(i, carry):
    j = 2 * i
    half(j, sidx0, semi0, rows0, sem0, sidx1, semi1, rows1, sem1)
    half(j + 1, sidx1, semi1, rows1, sem1, sidx0, semi0, rows0, sem0)
    return carry

  lax.fori_loop(0, NCHF // 2 - 1, body, 0)
  wait_sidx(sidx1, semi1)
  pltpu.async_copy(table_hbm.at[sidx1], rows1, sem1)
  pltpu.async_copy(src_hbm.at[pl.ds(ebase + NCHF * C, CT)], sidxt, semi0)
  pltpu.make_async_copy(table_hbm.at[sidx0], rows0, sem0).wait()
  pltpu.sync_copy(rows0, acc.at[didx(NCHF - 2)], add=True)
  pltpu.make_async_copy(src_hbm.at[pl.ds(ebase, CT)], sidxt, semi0).wait()
  pltpu.async_copy(table_hbm.at[sidxt], rowst, sem0)
  pltpu.make_async_copy(table_hbm.at[sidx1], rows1, sem1).wait()
  pltpu.sync_copy(rows1, acc.at[didx(NCHF - 1)], add=True)
  pltpu.make_async_copy(table_hbm.at[sidxt], rowst, sem0).wait()
  pltpu.sync_copy(rowst, acc.at[didx_all.at[pl.ds(NCHF * C, CT)]], add=True)

  plsc.subcore_barrier()
  pltpu.sync_copy(acc.at[pl.ds(sid * RPT, RPT)],
                  out_hbm.at[cid, pl.ds(sid * RPT, RPT)])


@functools.cache
def _scat_call():
  return pl.kernel(
      _scat_body,
      out_type=jax.ShapeDtypeStruct((NC, NP, H), jnp.float32),
      mesh=_mesh(),
      scratch_types=[
          pltpu.VMEM((C,), jnp.int32),
          pltpu.VMEM((C,), jnp.int32),
          pltpu.VMEM((CT,), jnp.int32),
          pltpu.VMEM((EPW,), jnp.int32),
          pltpu.VMEM((C, H), jnp.float32),
          pltpu.VMEM((C, H), jnp.float32),
          pltpu.VMEM((CT, H), jnp.float32),
          pltpu.VMEM_SHARED((NP, H), jnp.float32),
          pltpu.SemaphoreType.DMA,
          pltpu.SemaphoreType.DMA,
          pltpu.SemaphoreType.DMA,
          pltpu.SemaphoreType.DMA,
      ],
  )


def _dinv(d_ref):
  return lax.rsqrt(d_ref[...] + 1.0)


def _tc1_body(x_ref, w1_ref, d_ref, hs1_ref):
  h = jnp.dot(x_ref[...], w1_ref[...], preferred_element_type=jnp.float32)
  hs1_ref[...] = h * _dinv(d_ref)


def _tc1_call(x, w1, d):
  return pl.pallas_call(
      _tc1_body,
      out_shape=jax.ShapeDtypeStruct((N, H), jnp.float32),
  )(x, w1, d)


def _tc2_body(agg_ref, hs1_ref, d_ref, w2_ref, b1_ref, hs2_ref):
  dinv = _dinv(d_ref)
  agg = agg_ref[0, :N, :] + agg_ref[1, :N, :]
  out1 = (agg + hs1_ref[...]) * dinv + b1_ref[...]
  h1 = jnp.maximum(out1, 0.0)
  hs2_ref[...] = jnp.dot(h1, w2_ref[...],
                         preferred_element_type=jnp.float32) * dinv


def _tc2_call(agg1, hs1, d, w2, b1r):
  return pl.pallas_call(
      _tc2_body,
      out_shape=jax.ShapeDtypeStruct((N, H), jnp.float32),
  )(agg1, hs1, d, w2, b1r)


def _tc3_body(agg_ref, hs2_ref, d_ref, b2_ref, batch_ref, pcap_ref,
              wc_ref, bc_ref, wot_ref, bo_ref, wpt_ref, bp_ref,
              orig_ref, proc_ref):
  dinv = _dinv(d_ref)
  agg = agg_ref[0, :N, :] + agg_ref[1, :N, :]
  h2 = (agg + hs2_ref[...]) * dinv + b2_ref[...]
  ids = lax.broadcasted_iota(jnp.int32, (G, N), 0)
  oh = jnp.where(batch_ref[...] == ids, 1.0, 0.0)
  sums = jnp.dot(oh, h2, preferred_element_type=jnp.float32)
  counts = jnp.sum(oh, axis=1, keepdims=True)
  ge = sums / jnp.maximum(counts, 1.0)
  pe = jnp.dot(pcap_ref[...], wc_ref[...],
               preferred_element_type=jnp.float32) + bc_ref[...]
  comb = jnp.concatenate([ge, pe], axis=1)
  orig_ref[...] = jnp.dot(comb, wot_ref[...],
                          preferred_element_type=jnp.float32) + bo_ref[...]
  proc_ref[...] = jnp.dot(comb, wpt_ref[...],
                          preferred_element_type=jnp.float32) + bp_ref[...]


def _tc3_call(agg2, hs2, d, b2r, batch_r, pcap, wc, bcr, wot, bor, wpt,
              bpr):
  return pl.pallas_call(
      _tc3_body,
      out_shape=[
          jax.ShapeDtypeStruct((G, NIPS), jnp.float32),
          jax.ShapeDtypeStruct((G, NPROC), jnp.float32),
      ],
  )(agg2, hs2, d, b2r, batch_r, pcap, wc, bcr, wot, bor, wpt, bpr)


@jax.jit
def kernel(x, edge_index, batch, pcap_features, W1, b1, W2, b2, Wc, bc,
           Wo, bo, Wp, bp):
  src = edge_index[0]
  dst = edge_index[1]
  zeros_h = jnp.zeros((RPT, H), jnp.float32)

  degf = _deg_call()(dst)
  d = (degf[:N] + degf[NP:NP + N]).reshape(N, 1)
  hs1 = _tc1_call(x, W1, d)
  agg1 = _scat_call()(src, dst, hs1, zeros_h)
  hs2 = _tc2_call(agg1, hs1, d, W2, b1[None, :])
  agg2 = _scat_call()(src, dst, hs2, zeros_h)
  origin, process = _tc3_call(
      agg2, hs2, d, b2[None, :], batch.reshape(1, N), pcap_features,
      Wc[:, :, 1].T, bc[None, :], Wo.T, bo[None, :], Wp.T, bp[None, :])
  return (origin, process)


# speedup vs baseline: 1.1163x; 1.0381x over previous
import functools

import jax
import jax.numpy as jnp
from jax import lax
from jax.experimental import pallas as pl
from jax.experimental.pallas import tpu as pltpu
from jax.experimental.pallas import tpu_sc as plsc

N = 10000
E = 320000
SVG = 128
PCAP = 64
H = 128
NPROC = 128
NIPS = 1024
G = 64

NC = 2
NS = 16
NW = NC * NS

C = 128
EPW = E // NW
NCHF = EPW // C
CT = EPW - NCHF * C

NP = 10240
RPT = NP // NS


def _mesh():
  return plsc.VectorSubcoreMesh(
      core_axis_name="c", subcore_axis_name="s", num_cores=NC, num_subcores=NS
  )


def _deg_body(dst_hbm, out_hbm, didx_all, ones_v, zeros_v, acc, sem):
  cid = lax.axis_index("c")
  sid = lax.axis_index("s")
  wid = cid * NS + sid
  ebase = wid * EPW
  pltpu.async_copy(dst_hbm.at[pl.ds(ebase, EPW)], didx_all, sem)

  def fill(i, carry):
    zeros_v[pl.ds(i * 16, 16)] = jnp.zeros((16,), jnp.float32)
    return carry

  lax.fori_loop(0, RPT // 16, fill, 0)

  def fill1(i, carry):
    ones_v[pl.ds(i * 16, 16)] = jnp.ones((16,), jnp.float32)
    return carry

  lax.fori_loop(0, C // 16, fill1, 0)

  pltpu.sync_copy(zeros_v, acc.at[pl.ds(sid * RPT, RPT)])
  pltpu.make_async_copy(dst_hbm.at[pl.ds(0, EPW)], didx_all, sem).wait()
  plsc.subcore_barrier()

  def body(i, carry):
    pltpu.sync_copy(ones_v, acc.at[didx_all.at[pl.ds(i * C, C)]], add=True)
    return carry

  lax.fori_loop(0, NCHF, body, 0)
  pltpu.sync_copy(ones_v.at[pl.ds(0, CT)],
                  acc.at[didx_all.at[pl.ds(NCHF * C, CT)]], add=True)

  plsc.subcore_barrier()
  pltpu.sync_copy(acc.at[pl.ds(sid * RPT, RPT)],
                  out_hbm.at[pl.ds(cid * NP + sid * RPT, RPT)])


@functools.cache
def _deg_call():
  return pl.kernel(
      _deg_body,
      out_type=jax.ShapeDtypeStruct((NC * NP,), jnp.float32),
      mesh=_mesh(),
      scratch_types=[
          pltpu.VMEM((EPW,), jnp.int32),
          pltpu.VMEM((C,), jnp.float32),
          pltpu.VMEM((RPT,), jnp.float32),
          pltpu.VMEM_SHARED((NP,), jnp.float32),
          pltpu.SemaphoreType.DMA,
      ],
  )


def _scat_body(src_hbm, dst_hbm, table_hbm, out_hbm,
               sidx0, sidx1, sidxt, didx_all, rows0, rows1, rowst, zbuf, acc,
               semi0, semi1, sem0, sem1):
  cid = lax.axis_index("c")
  sid = lax.axis_index("s")
  wid = cid * NS + sid
  ebase = wid * EPW

  def load_sidx(j, sidx, semi):
    pltpu.async_copy(src_hbm.at[pl.ds(ebase + j * C, C)], sidx, semi)

  def wait_sidx(sidx, semi):
    pltpu.make_async_copy(src_hbm.at[pl.ds(ebase, C)], sidx, semi).wait()

  def didx(j):
    return didx_all.at[pl.ds(j * C, C)]

  pltpu.async_copy(dst_hbm.at[pl.ds(ebase, EPW)], didx_all, semi0)
  load_sidx(0, sidx0, semi0)
  load_sidx(1, sidx1, semi1)

  def zfill(i, carry):
    zbuf[i // (H // 16), pl.ds((i % (H // 16)) * 16, 16)] = jnp.zeros(
        (16,), jnp.float32)
    return carry

  lax.fori_loop(0, 16 * (H // 16), zfill, 0)
  wait_sidx(sidx0, semi0)
  pltpu.async_copy(table_hbm.at[sidx0], rows0, sem0)
  for q in range(RPT // 16):
    pltpu.sync_copy(zbuf, acc.at[pl.ds(sid * RPT + q * 16, 16)])
  pltpu.make_async_copy(dst_hbm.at[pl.ds(0, EPW)], didx_all, semi0).wait()
  plsc.subcore_barrier()

  def half(j, sidx_a, semi_a, rows_a, sem_a, sidx_b, semi_b, rows_b, sem_b):
    wait_sidx(sidx_b, semi_b)
    pltpu.async_copy(table_hbm.at[sidx_b], rows_b, sem_b)
    pltpu.make_async_copy(table_hbm.at[sidx_a], rows_a, sem_a).wait()
    pltpu.sync_copy(rows_a, acc.at[didx(j)], add=True)
    load_sidx(j + 2, sidx_a, semi_a)

  def body(i, carry):
    j = 2 * i
    half(j, sidx0, semi0, rows0, sem0, sidx1, semi1, rows1, sem1)
    half(j + 1, sidx1, semi1, rows1, sem1, sidx0, semi0, rows0, sem0)
    return carry

  lax.fori_loop(0, NCHF // 2 - 1, body, 0)
  wait_sidx(sidx1, semi1)
  pltpu.async_copy(table_hbm.at[sidx1], rows1, sem1)
  pltpu.async_copy(src_hbm.at[pl.ds(ebase + NCHF * C, CT)], sidxt, semi0)
  pltpu.make_async_copy(table_hbm.at[sidx0], rows0, sem0).wait()
  pltpu.sync_copy(rows0, acc.at[didx(NCHF - 2)], add=True)
  pltpu.make_async_copy(src_hbm.at[pl.ds(ebase, CT)], sidxt, semi0).wait()
  pltpu.async_copy(table_hbm.at[sidxt], rowst, sem0)
  pltpu.make_async_copy(table_hbm.at[sidx1], rows1, sem1).wait()
  pltpu.sync_copy(rows1, acc.at[didx(NCHF - 1)], add=True)
  pltpu.make_async_copy(table_hbm.at[sidxt], rowst, sem0).wait()
  pltpu.sync_copy(rowst, acc.at[didx_all.at[pl.ds(NCHF * C, CT)]], add=True)

  plsc.subcore_barrier()
  pltpu.sync_copy(acc.at[pl.ds(sid * RPT, RPT)],
                  out_hbm.at[cid, pl.ds(sid * RPT, RPT)])


@functools.cache
def _scat_call():
  return pl.kernel(
      _scat_body,
      out_type=jax.ShapeDtypeStruct((NC, NP, H), jnp.float32),
      mesh=_mesh(),
      scratch_types=[
          pltpu.VMEM((C,), jnp.int32),
          pltpu.VMEM((C,), jnp.int32),
          pltpu.VMEM((CT,), jnp.int32),
          pltpu.VMEM((EPW,), jnp.int32),
          pltpu.VMEM((C, H), jnp.float32),
          pltpu.VMEM((C, H), jnp.float32),
          pltpu.VMEM((CT, H), jnp.float32),
          pltpu.VMEM((16, H), jnp.float32),
          pltpu.VMEM_SHARED((NP, H), jnp.float32),
          pltpu.SemaphoreType.DMA,
          pltpu.SemaphoreType.DMA,
          pltpu.SemaphoreType.DMA,
          pltpu.SemaphoreType.DMA,
      ],
  )


def _dinv(d_ref):
  return lax.rsqrt(d_ref[...] + 1.0)


def _tc1_body(x_ref, w1_ref, d_ref, hs1_ref):
  h = jnp.dot(x_ref[...], w1_ref[...], preferred_element_type=jnp.float32)
  hs1_ref[...] = h * _dinv(d_ref)


def _tc1_call(x, w1, d):
  return pl.pallas_call(
      _tc1_body,
      out_shape=jax.ShapeDtypeStruct((N, H), jnp.float32),
  )(x, w1, d)


def _tc2_body(agg_ref, hs1_ref, d_ref, w2_ref, b1_ref, hs2_ref):
  dinv = _dinv(d_ref)
  agg = agg_ref[0, :N, :] + agg_ref[1, :N, :]
  out1 = (agg + hs1_ref[...]) * dinv + b1_ref[...]
  h1 = jnp.maximum(out1, 0.0)
  hs2_ref[...] = jnp.dot(h1, w2_ref[...],
                         preferred_element_type=jnp.float32) * dinv


def _tc2_call(agg1, hs1, d, w2, b1r):
  return pl.pallas_call(
      _tc2_body,
      out_shape=jax.ShapeDtypeStruct((N, H), jnp.float32),
  )(agg1, hs1, d, w2, b1r)


def _tc3_body(agg_ref, hs2_ref, d_ref, b2_ref, batch_ref, pcap_ref,
              wc_ref, bc_ref, wot_ref, bo_ref, wpt_ref, bp_ref,
              orig_ref, proc_ref):
  dinv = _dinv(d_ref)
  agg = agg_ref[0, :N, :] + agg_ref[1, :N, :]
  h2 = (agg + hs2_ref[...]) * dinv + b2_ref[...]
  ids = lax.broadcasted_iota(jnp.int32, (G, N), 0)
  oh = jnp.where(batch_ref[...] == ids, 1.0, 0.0)
  sums = jnp.dot(oh, h2, preferred_element_type=jnp.float32)
  counts = jnp.sum(oh, axis=1, keepdims=True)
  ge = sums / jnp.maximum(counts, 1.0)
  pe = jnp.dot(pcap_ref[...], wc_ref[...],
               preferred_element_type=jnp.float32) + bc_ref[...]
  comb = jnp.concatenate([ge, pe], axis=1)
  orig_ref[...] = jnp.dot(comb, wot_ref[...],
                          preferred_element_type=jnp.float32) + bo_ref[...]
  proc_ref[...] = jnp.dot(comb, wpt_ref[...],
                          preferred_element_type=jnp.float32) + bp_ref[...]


def _tc3_call(agg2, hs2, d, b2r, batch_r, pcap, wc, bcr, wot, bor, wpt,
              bpr):
  return pl.pallas_call(
      _tc3_body,
      out_shape=[
          jax.ShapeDtypeStruct((G, NIPS), jnp.float32),
          jax.ShapeDtypeStruct((G, NPROC), jnp.float32),
      ],
  )(agg2, hs2, d, b2r, batch_r, pcap, wc, bcr, wot, bor, wpt, bpr)


@jax.jit
def kernel(x, edge_index, batch, pcap_features, W1, b1, W2, b2, Wc, bc,
           Wo, bo, Wp, bp):
  src = edge_index[0]
  dst = edge_index[1]
  degf = _deg_call()(dst)
  d = (degf[:N] + degf[NP:NP + N]).reshape(N, 1)
  hs1 = _tc1_call(x, W1, d)
  agg1 = _scat_call()(src, dst, hs1)
  hs2 = _tc2_call(agg1, hs1, d, W2, b1[None, :])
  agg2 = _scat_call()(src, dst, hs2)
  origin, process = _tc3_call(
      agg2, hs2, d, b2[None, :], batch.reshape(1, N), pcap_features,
      Wc[:, :, 1].T, bc[None, :], Wo.T, bo[None, :], Wp.T, bp[None, :])
  return (origin, process)
